# baseline (speedup 1.0000x reference)
import functools

import jax
import jax.numpy as jnp
from jax import lax
from jax.experimental import pallas as pl
from jax.experimental.pallas import tpu as pltpu
from jax.experimental.pallas import tpu_sc as plsc

N = 16384
K = 1024
D = 64
BN = 2048

NW = 32
CHUNK = 128


def _index_body(w_ref, xt_ref, wt_ref, idx_ref):
    Wm = w_ref[...]
    xtv = xt_ref[...]
    wtv = wt_ref[...]

    s = lax.dot_general(Wm * -2.0, xtv, (((1,), (0,)), ((), ())),
                        preferred_element_type=jnp.float32)

    y = xtv * xtv
    p = y[0:8, :]
    for k in range(1, 8):
        p = p + y[8 * k:8 * k + 8, :]
    t1 = p[0:4, :] + p[4:8, :]
    t2 = t1[0:2, :] + t1[2:4, :]
    x2 = t2[0:1, :] + t2[1:2, :]

    w2col = jnp.sum(wtv * wtv, axis=0, keepdims=True).T

    dist = jnp.sqrt(jnp.maximum((x2 + s) + w2col, 0.0))
    m = jnp.min(dist, axis=0, keepdims=True)
    iota = lax.broadcasted_iota(jnp.int32, (K, 1), 0).astype(jnp.float32)
    cand = jnp.where(dist == m, iota, float(K))
    idxf = jnp.min(cand, axis=0, keepdims=True)
    idx_ref[...] = idxf.astype(jnp.int32).reshape(1, 1, BN)


def _compute_indices(W, xt, wt, n):
    nb = n // BN
    out = pl.pallas_call(
        _index_body,
        grid=(nb,),
        in_specs=[
            pl.BlockSpec((K, D), lambda i: (0, 0)),
            pl.BlockSpec((D, BN), lambda i: (0, i)),
            pl.BlockSpec((D, K), lambda i: (0, 0)),
        ],
        out_specs=pl.BlockSpec((1, 1, BN), lambda i: (i, 0, 0)),
        out_shape=jax.ShapeDtypeStruct((nb, 1, BN), jnp.int32),
    )(W, xt, wt)
    return out.reshape(n)


def _gather_rows(Wp, idx3, n):
    bpw = n // NW
    nch = bpw // CHUNK
    mesh = plsc.VectorSubcoreMesh(core_axis_name="c", subcore_axis_name="s")

    @functools.partial(
        pl.kernel,
        mesh=mesh,
        out_type=jax.ShapeDtypeStruct((n, 2 * D), jnp.float32),
        scratch_types=[
            pltpu.VMEM((nch, CHUNK), jnp.int32),
            pltpu.VMEM((bpw, 2 * D), jnp.float32),
            pltpu.SemaphoreType.DMA,
        ],
    )
    def gk(table_hbm, idx_hbm, out_hbm, idx_v, rows_v, sem):
        wid = lax.axis_index("s") * 2 + lax.axis_index("c")
        base = wid * bpw
        pltpu.sync_copy(idx_hbm.at[wid], idx_v)
        cps = [
            pltpu.async_copy(
                table_hbm.at[idx_v.at[j]],
                rows_v.at[pl.ds(j * CHUNK, CHUNK)],
                sem,
            )
            for j in range(nch)
        ]
        for c in cps:
            c.wait()
        pltpu.sync_copy(rows_v, out_hbm.at[pl.ds(base, bpw)])

    return gk(Wp, idx3)


def kernel(x, W):
    indices = _compute_indices(W, x.T, W.T, N)
    Wp = jnp.concatenate([W, jnp.zeros((K, D), jnp.float32)], axis=1)
    zp = _gather_rows(Wp, indices.reshape(NW, N // NW // CHUNK, CHUNK), N)
    z = lax.slice(zp, (0, 0), (N, D))
    return (z, z, x, indices)


# speedup vs baseline: 1.0307x; 1.0307x over previous
import functools

import jax
import jax.numpy as jnp
from jax import lax
from jax.experimental import pallas as pl
from jax.experimental.pallas import tpu as pltpu
from jax.experimental.pallas import tpu_sc as plsc

N = 16384
K = 1024
D = 64
BN = 4096

NW = 32
CHUNK = 128


def _index_body(w_ref, xt_ref, wt_ref, idx_ref):
    Wm = w_ref[...]
    xtv = xt_ref[...]
    wtv = wt_ref[...]

    s = lax.dot_general(Wm * -2.0, xtv, (((1,), (0,)), ((), ())),
                        preferred_element_type=jnp.float32)

    y = xtv * xtv
    p = y[0:8, :]
    for k in range(1, 8):
        p = p + y[8 * k:8 * k + 8, :]
    t1 = p[0:4, :] + p[4:8, :]
    t2 = t1[0:2, :] + t1[2:4, :]
    x2 = t2[0:1, :] + t2[1:2, :]

    w2col = jnp.sum(wtv * wtv, axis=0, keepdims=True).T

    dist = jnp.sqrt(jnp.maximum((x2 + s) + w2col, 0.0))
    m = jnp.min(dist, axis=0, keepdims=True)
    iota = lax.broadcasted_iota(jnp.int32, (K, 1), 0).astype(jnp.float32)
    cand = jnp.where(dist == m, iota, float(K))
    idxf = jnp.min(cand, axis=0, keepdims=True)
    idx_ref[...] = idxf.astype(jnp.int32).reshape(1, 1, BN)


def _compute_indices(W, xt, wt, n):
    nb = n // BN
    out = pl.pallas_call(
        _index_body,
        grid=(nb,),
        in_specs=[
            pl.BlockSpec((K, D), lambda i: (0, 0)),
            pl.BlockSpec((D, BN), lambda i: (0, i)),
            pl.BlockSpec((D, K), lambda i: (0, 0)),
        ],
        out_specs=pl.BlockSpec((1, 1, BN), lambda i: (i, 0, 0)),
        out_shape=jax.ShapeDtypeStruct((nb, 1, BN), jnp.int32),
    )(W, xt, wt)
    return out.reshape(n)


def _gather_rows(Wp, idx3, n):
    bpw = n // NW
    nch = bpw // CHUNK
    mesh = plsc.VectorSubcoreMesh(core_axis_name="c", subcore_axis_name="s")

    @functools.partial(
        pl.kernel,
        mesh=mesh,
        out_type=jax.ShapeDtypeStruct((n, 2 * D), jnp.float32),
        scratch_types=[
            pltpu.VMEM((nch, CHUNK), jnp.int32),
            pltpu.VMEM((bpw, 2 * D), jnp.float32),
            pltpu.SemaphoreType.DMA,
        ],
    )
    def gk(table_hbm, idx_hbm, out_hbm, idx_v, rows_v, sem):
        wid = lax.axis_index("s") * 2 + lax.axis_index("c")
        base = wid * bpw
        pltpu.sync_copy(idx_hbm.at[wid], idx_v)
        cps = [
            pltpu.async_copy(
                table_hbm.at[idx_v.at[j]],
                rows_v.at[pl.ds(j * CHUNK, CHUNK)],
                sem,
            )
            for j in range(nch)
        ]
        for c in cps:
            c.wait()
        pltpu.sync_copy(rows_v, out_hbm.at[pl.ds(base, bpw)])

    return gk(Wp, idx3)


def kernel(x, W):
    indices = _compute_indices(W, x.T, W.T, N)
    Wp = jnp.concatenate([W, jnp.zeros((K, D), jnp.float32)], axis=1)
    zp = _gather_rows(Wp, indices.reshape(NW, N // NW // CHUNK, CHUNK), N)
    z = lax.slice(zp, (0, 0), (N, D))
    return (z, z, x, indices)
